# speedup vs baseline: 5.1707x; 5.1707x over previous
import functools

import jax
import jax.numpy as jnp
from jax import lax
from jax.experimental import pallas as pl
from jax.experimental.pallas import tpu as pltpu
from jax.experimental.pallas import tpu_sc as plsc

NC = 2
NS = 16
NW = NC * NS

D = 128
CHUNK = 128


@functools.partial(jax.jit, static_argnames=("b_per_w", "n_chunks"))
def _embed_lookup(idx_flat, table, *, b_per_w, n_chunks):
    B = idx_flat.shape[0]

    mesh = plsc.VectorSubcoreMesh(
        core_axis_name="c", subcore_axis_name="s", num_cores=NC, num_subcores=NS
    )

    @functools.partial(
        pl.kernel,
        out_type=jax.ShapeDtypeStruct((B, D), jnp.float32),
        mesh=mesh,
        scratch_types=[
            pltpu.VMEM((CHUNK,), jnp.int32),
            pltpu.VMEM((CHUNK, D), jnp.float32),
            pltpu.SemaphoreType.DMA,
        ],
    )
    def body(idx_hbm, table_hbm, out_hbm, idx_v, rows_v, sem):
        wid = lax.axis_index("s") * NC + lax.axis_index("c")
        base = wid * b_per_w

        def step(i, carry):
            off = base + i * CHUNK
            pltpu.sync_copy(idx_hbm.at[pl.ds(off, CHUNK)], idx_v)
            pltpu.async_copy(table_hbm.at[idx_v], rows_v, sem).wait()
            pltpu.sync_copy(rows_v, out_hbm.at[pl.ds(off, CHUNK)])
            return carry

        lax.fori_loop(0, n_chunks, step, 0)

    return body(idx_flat, table)


def kernel(input_ids, table):
    Bt, T = input_ids.shape
    B = Bt * T
    idx_flat = input_ids.reshape(B).astype(jnp.int32)
    assert B % (NW * CHUNK) == 0
    b_per_w = B // NW
    out = _embed_lookup(idx_flat, table, b_per_w=b_per_w, n_chunks=b_per_w // CHUNK)
    return out.reshape(Bt, T, D)


# speedup vs baseline: 9.1958x; 1.7785x over previous
import functools

import jax
import jax.numpy as jnp
from jax import lax
from jax.experimental import pallas as pl
from jax.experimental.pallas import tpu as pltpu
from jax.experimental.pallas import tpu_sc as plsc

NC = 2
NS = 16
NW = NC * NS

D = 128
CHUNK = 128
NBUF = 4


@functools.partial(jax.jit, static_argnames=("b_per_w", "n_chunks"))
def _embed_lookup(idx_grp, table, *, b_per_w, n_chunks):
    B = NW * b_per_w

    mesh = plsc.VectorSubcoreMesh(
        core_axis_name="c", subcore_axis_name="s", num_cores=NC, num_subcores=NS
    )

    @functools.partial(
        pl.kernel,
        out_type=jax.ShapeDtypeStruct((B, D), jnp.float32),
        mesh=mesh,
        scratch_types=[
            pltpu.VMEM((n_chunks, CHUNK), jnp.int32),
            pltpu.VMEM((NBUF, CHUNK, D), jnp.float32),
            pltpu.SemaphoreType.DMA((NBUF,)),
            pltpu.SemaphoreType.DMA((NBUF,)),
        ],
    )
    def body(idx_hbm, table_hbm, out_hbm, idx_v, rows_v, gsem, ssem):
        wid = lax.axis_index("s") * NC + lax.axis_index("c")
        base = wid * b_per_w
        pltpu.sync_copy(idx_hbm.at[wid], idx_v)

        def gather(g, b):
            return pltpu.make_async_copy(
                table_hbm.at[idx_v.at[g]], rows_v.at[b], gsem.at[b]
            )

        def store(g, b):
            return pltpu.make_async_copy(
                rows_v.at[b], out_hbm.at[pl.ds(base + g * CHUNK, CHUNK)], ssem.at[b]
            )

        n_outer = (n_chunks + 2 * NBUF - 1) // NBUF

        def outer(it, carry):
            for b in range(NBUF):
                g = it * NBUF + b

                @pl.when(jnp.logical_and(g >= NBUF, g < n_chunks))
                def _():
                    store(g - NBUF, b).wait()

                @pl.when(g < n_chunks)
                def _():
                    gather(g, b).start()

                gs = g - (NBUF - 1)
                b2 = (b + 1) % NBUF

                @pl.when(jnp.logical_and(gs >= 0, gs < n_chunks))
                def _():
                    gather(gs, b2).wait()
                    store(gs, b2).start()

            return carry

        lax.fori_loop(0, n_outer, outer, 0)

        for g in range(n_chunks - NBUF, n_chunks):
            store(g, g % NBUF).wait()

    return body(idx_grp, table)


def kernel(input_ids, table):
    Bt, T = input_ids.shape
    B = Bt * T
    assert B % (NW * CHUNK) == 0
    b_per_w = B // NW
    n_chunks = b_per_w // CHUNK
    idx_grp = input_ids.reshape(NW, n_chunks, CHUNK).astype(jnp.int32)
    out = _embed_lookup(idx_grp, table, b_per_w=b_per_w, n_chunks=n_chunks)
    return out.reshape(Bt, T, D)
